# baseline (speedup 1.0000x reference)
import functools

import jax
import jax.numpy as jnp
from jax import lax
from jax.experimental import pallas as pl
from jax.experimental.pallas import tpu as pltpu
from jax.experimental.pallas import tpu_sc as plsc


def _matmul_body(x1_ref, x2_ref, w_ref, b_ref, o_ref):
    bm = x1_ref.shape[0]
    o_ref[:bm, :] = (
        jnp.dot(x1_ref[...], w_ref[...], preferred_element_type=jnp.float32)
        + b_ref[...]
    )
    o_ref[bm:, :] = (
        jnp.dot(x2_ref[...], w_ref[...], preferred_element_type=jnp.float32)
        + b_ref[...]
    )


@functools.lru_cache(maxsize=None)
def _make_logits(T, H, Rp, BM):
    return pl.pallas_call(
        _matmul_body,
        grid=(T // (2 * BM),),
        in_specs=[
            pl.BlockSpec((BM, H), lambda i: (2 * i, 0)),
            pl.BlockSpec((BM, H), lambda i: (2 * i + 1, 0)),
            pl.BlockSpec((H, Rp), lambda i: (0, 0)),
            pl.BlockSpec((1, Rp), lambda i: (0, 0)),
        ],
        out_specs=pl.BlockSpec((2 * BM, Rp), lambda i: (i, 0)),
        out_shape=jax.ShapeDtypeStruct((T, Rp), jnp.float32),
        compiler_params=pltpu.CompilerParams(
            dimension_semantics=("parallel",)
        ),
    )


@functools.lru_cache(maxsize=None)
def _make_gather(B, D):
    info = plsc.get_sparse_core_info()
    NC, NS = info.num_cores, info.num_subcores
    NW = NC * NS
    b_per_w = B // NW
    mesh = plsc.VectorSubcoreMesh(core_axis_name="c", subcore_axis_name="s")

    @functools.partial(
        pl.kernel,
        mesh=mesh,
        out_type=jax.ShapeDtypeStruct((B, D), jnp.float32),
        scratch_types=[
            pltpu.VMEM((b_per_w,), jnp.int32),
            pltpu.VMEM((b_per_w, D), jnp.float32),
            pltpu.SemaphoreType.DMA,
        ],
    )
    def gather_k(table_hbm, idx_hbm, out_hbm, idx_v, rows_v, sem):
        wid = lax.axis_index("s") * NC + lax.axis_index("c")
        base = wid * b_per_w
        pltpu.sync_copy(idx_hbm.at[pl.ds(base, b_per_w)], idx_v)
        pltpu.async_copy(table_hbm.at[idx_v], rows_v, sem).wait()
        pltpu.sync_copy(rows_v, out_hbm.at[pl.ds(base, b_per_w)])

    return gather_k


@jax.jit
def kernel(x, scope, query, rel_mat, bias):
    T, H = x.shape
    B = scope.shape[0]
    R = rel_mat.shape[1]
    Rp = 128

    w = jnp.zeros((H, Rp), jnp.float32).at[:, :R].set(rel_mat)
    b2 = jnp.zeros((1, Rp), jnp.float32).at[0, :R].set(bias)

    logits = _make_logits(T, H, Rp, 1024)(x, x, w, b2)

    starts = jnp.clip(scope[:, 0], 0, T - 1).astype(jnp.int32)
    out = _make_gather(B, Rp)(logits, starts)
    return out[:, :R]
